# baseline (speedup 1.0000x reference)
import jax
import jax.numpy as jnp
from jax import lax
from jax.experimental import pallas as pl
from jax.experimental.pallas import tpu as pltpu
from jax.experimental.pallas import tpu_sc as plsc

N_USERS = 4096
N_ITEMS = 1000
H = 12
A = 1 << H
R = 256
NW = 32
UPW = N_USERS // NW
SG = 32
NSG = UPW // SG
G = 16
WORDS = 250


def _eb_body(dist_ref, eb_ref):
    dist = dist_ref[...]
    k_ids = lax.broadcasted_iota(jnp.int32, (A, 128), 0)
    j_ids = jnp.minimum(lax.broadcasted_iota(jnp.int32, (A, 128), 1), 31)
    bitmat = ((k_ids >> j_ids) & 1).astype(jnp.float32)
    eb = jnp.dot(dist, bitmat, preferred_element_type=jnp.float32)
    eb_ref[...] = eb[:, :16]


def _sc_body(eb_hbm, hist_hbm, o_hbm, act_hbm, num_hbm,
             eb_v, hist_v, obuf, wbuf, num_v):
    wid = lax.axis_index("s") * 2 + lax.axis_index("c")
    base = wid * UPW
    pltpu.sync_copy(eb_hbm.at[pl.ds(base, UPW), :], eb_v)
    pltpu.sync_copy(hist_hbm.at[pl.ds(base, UPW), :], hist_v)

    zf = jnp.zeros((G,), jnp.float32)
    zi = jnp.zeros((G,), jnp.int32)
    rows16 = lax.broadcasted_iota(jnp.int32, (G,), 0)

    def zero_o_row(u, carry):
        def zero_chunk(i, c2):
            obuf[u, pl.ds(jnp.minimum(i * 16, N_ITEMS - 16), 16)] = zf
            return c2
        return lax.fori_loop(0, N_ITEMS // 16 + 1, zero_chunk, carry)

    lax.fori_loop(0, SG, zero_o_row, 0)

    def zero_w_row(u, carry):
        def zero_chunk(i, c2):
            wbuf[u, pl.ds(jnp.minimum(i * 16, WORDS - 16), 16)] = zi
            return c2
        return lax.fori_loop(0, WORDS // 16 + 1, zero_chunk, carry)

    lax.fori_loop(0, SG, zero_w_row, 0)

    def supergroup(sg, carry):
        for g2 in range(2):
            urows = sg * SG + g2 * G + rows16
            brows = g2 * G + rows16

            def hcol(j):
                return plsc.load_gather(
                    hist_v, [urows, jnp.full((G,), j, jnp.int32)]
                )

            for j in range(H):
                col = hcol(j)
                val = plsc.load_gather(
                    eb_v, [urows, jnp.full((G,), j, jnp.int32)]
                )
                plsc.addupdate_scatter(obuf, [brows, col], val)
            for j in range(H):
                col = hcol(j)
                oval = plsc.load_gather(obuf, [brows, col])
                bit = (oval > 0.5).astype(jnp.int32)
                widx = col >> 2
                sh = (col & 3) * 8
                wold = plsc.load_gather(wbuf, [brows, widx])
                wnew = (
                    wold & jnp.bitwise_not(jnp.left_shift(jnp.int32(255), sh))
                ) | (bit << sh)
                plsc.store_scatter(wbuf, [brows, widx], wnew)
            num = jnp.zeros((G,), jnp.int32)
            for c in range(H):
                oval = plsc.load_gather(
                    obuf, [brows, jnp.full((G,), c, jnp.int32)]
                )
                num = num | ((oval > 0.5).astype(jnp.int32) << c)
            num_v[pl.ds(sg * SG + g2 * G, G)] = num

        u0 = base + sg * SG
        pltpu.sync_copy(obuf, o_hbm.at[pl.ds(u0, SG), :])
        pltpu.sync_copy(wbuf, act_hbm.at[pl.ds(u0, SG), :])

        for g2 in range(2):
            urows = sg * SG + g2 * G + rows16
            brows = g2 * G + rows16
            for j in range(H):
                col = plsc.load_gather(
                    hist_v, [urows, jnp.full((G,), j, jnp.int32)]
                )
                plsc.store_scatter(obuf, [brows, col], zf)
                plsc.store_scatter(wbuf, [brows, col >> 2], zi)
        return carry

    lax.fori_loop(0, NSG, supergroup, 0)
    pltpu.sync_copy(num_v, num_hbm.at[pl.ds(base, UPW)])


def _make_sc_call(interpret=False):
    mesh = plsc.VectorSubcoreMesh(
        core_axis_name="c", subcore_axis_name="s", num_cores=2, num_subcores=16
    )
    return pl.kernel(
        _sc_body,
        out_type=[
            jax.ShapeDtypeStruct((N_USERS, N_ITEMS), jnp.float32),
            jax.ShapeDtypeStruct((N_USERS, WORDS), jnp.int32),
            jax.ShapeDtypeStruct((N_USERS,), jnp.int32),
        ],
        mesh=mesh,
        scratch_types=[
            pltpu.VMEM((UPW, 16), jnp.float32),
            pltpu.VMEM((UPW, H), jnp.int32),
            pltpu.VMEM((SG, N_ITEMS), jnp.float32),
            pltpu.VMEM((SG, WORDS), jnp.int32),
            pltpu.VMEM((UPW,), jnp.int32),
        ],
        compiler_params=pltpu.CompilerParams(needs_layout_passes=False),
        interpret=interpret,
    )


def kernel(distribution, history):
    hist = history.astype(jnp.int32)
    eb = pl.pallas_call(
        _eb_body,
        grid=(N_USERS // R,),
        in_specs=[pl.BlockSpec((R, A), lambda i: (i, 0))],
        out_specs=pl.BlockSpec((R, 16), lambda i: (i, 0)),
        out_shape=jax.ShapeDtypeStruct((N_USERS, 16), jnp.float32),
    )(distribution)
    o, act_words, num = _make_sc_call()(eb, hist)
    act = (
        lax.bitcast_convert_type(act_words, jnp.uint8)
        .reshape(N_USERS, N_ITEMS)
        .astype(jnp.bool_)
    )
    return (o, act, num)


# speedup vs baseline: 1.0524x; 1.0524x over previous
import jax
import jax.numpy as jnp
from jax import lax
from jax.experimental import pallas as pl
from jax.experimental.pallas import tpu as pltpu
from jax.experimental.pallas import tpu_sc as plsc

N_USERS = 4096
N_ITEMS = 1000
H = 12
A = 1 << H
R = 256
NW = 32
UPW = N_USERS // NW
G = 16
NG = UPW // G


def _eb_body(dist_ref, eb_ref):
    dist = dist_ref[...]
    k_ids = lax.broadcasted_iota(jnp.int32, (A, 128), 0)
    j_ids = jnp.minimum(lax.broadcasted_iota(jnp.int32, (A, 128), 1), 31)
    bitmat = ((k_ids >> j_ids) & 1).astype(jnp.float32)
    eb = jnp.dot(dist, bitmat, preferred_element_type=jnp.float32)
    eb_ref[...] = eb[:, :16]


def _sc_body(eb_hbm, hist_hbm, o_hbm, eb_v, hist_v, obuf):
    wid = lax.axis_index("s") * 2 + lax.axis_index("c")
    base = wid * UPW
    pltpu.sync_copy(eb_hbm.at[pl.ds(base, UPW), :], eb_v)
    pltpu.sync_copy(hist_hbm.at[pl.ds(base, UPW), :], hist_v)

    zf = jnp.zeros((G,), jnp.float32)
    rows = lax.broadcasted_iota(jnp.int32, (G,), 0)

    def zero_row(u, carry):
        def zero_chunk(i, c2):
            obuf[u, pl.ds(jnp.minimum(i * 16, N_ITEMS - 16), 16)] = zf
            return c2
        return lax.fori_loop(0, N_ITEMS // 16 + 1, zero_chunk, carry)

    lax.fori_loop(0, G, zero_row, 0)

    def group(g, carry):
        urows = g * G + rows

        def hcol(j):
            return plsc.load_gather(hist_v, [urows, jnp.full((G,), j, jnp.int32)])

        for j in range(H):
            col = hcol(j)
            val = plsc.load_gather(eb_v, [urows, jnp.full((G,), j, jnp.int32)])
            plsc.addupdate_scatter(obuf, [rows, col], val)
        pltpu.sync_copy(obuf, o_hbm.at[pl.ds(base + g * G, G), :])
        for j in range(H):
            plsc.store_scatter(obuf, [rows, hcol(j)], zf)
        return carry

    lax.fori_loop(0, NG, group, 0)


def _act_body(o_ref, act_ref, num_ref):
    o = o_ref[...]
    act = o > 0.5
    act_ref[...] = act
    pw = (1 << lax.broadcasted_iota(jnp.int32, (R, H), 1)).astype(jnp.int32)
    num_ref[...] = jnp.sum(act[:, :H].astype(jnp.int32) * pw, axis=1)


def _make_sc_call(interpret=False):
    mesh = plsc.VectorSubcoreMesh(
        core_axis_name="c", subcore_axis_name="s", num_cores=2, num_subcores=16
    )
    return pl.kernel(
        _sc_body,
        out_type=jax.ShapeDtypeStruct((N_USERS, N_ITEMS), jnp.float32),
        mesh=mesh,
        scratch_types=[
            pltpu.VMEM((UPW, 16), jnp.float32),
            pltpu.VMEM((UPW, H), jnp.int32),
            pltpu.VMEM((G, N_ITEMS), jnp.float32),
        ],
        compiler_params=pltpu.CompilerParams(needs_layout_passes=False),
        interpret=interpret,
    )


def kernel(distribution, history):
    hist = history.astype(jnp.int32)
    eb = pl.pallas_call(
        _eb_body,
        grid=(N_USERS // R,),
        in_specs=[pl.BlockSpec((R, A), lambda i: (i, 0))],
        out_specs=pl.BlockSpec((R, 16), lambda i: (i, 0)),
        out_shape=jax.ShapeDtypeStruct((N_USERS, 16), jnp.float32),
    )(distribution)
    o = _make_sc_call()(eb, hist)
    act, num = pl.pallas_call(
        _act_body,
        grid=(N_USERS // R,),
        in_specs=[pl.BlockSpec((R, N_ITEMS), lambda i: (i, 0))],
        out_specs=[
            pl.BlockSpec((R, N_ITEMS), lambda i: (i, 0)),
            pl.BlockSpec((R,), lambda i: (i,)),
        ],
        out_shape=[
            jax.ShapeDtypeStruct((N_USERS, N_ITEMS), jnp.bool_),
            jax.ShapeDtypeStruct((N_USERS,), jnp.int32),
        ],
    )(o)
    return (o, act, num)


# speedup vs baseline: 1.1252x; 1.0692x over previous
import jax
import jax.numpy as jnp
from jax import lax
from jax.experimental import pallas as pl
from jax.experimental.pallas import tpu as pltpu
from jax.experimental.pallas import tpu_sc as plsc

N_USERS = 4096
N_ITEMS = 1000
H = 12
A = 1 << H
R = 256
NW = 32
UPW = N_USERS // NW
G = 16
NG = UPW // G


def _eb_body(dist_ref, eb_ref):
    dist = dist_ref[...]
    k_ids = lax.broadcasted_iota(jnp.int32, (A, 128), 0)
    j_ids = jnp.minimum(lax.broadcasted_iota(jnp.int32, (A, 128), 1), 31)
    bitmat = ((k_ids >> j_ids) & 1).astype(jnp.bfloat16)
    eb = jnp.dot(dist.astype(jnp.bfloat16), bitmat,
                 preferred_element_type=jnp.float32)
    eb_ref[...] = eb[:, :16]


def _sc_body(eb_hbm, hist_hbm, o_hbm, eb_v, hist_v, obuf, sems):
    wid = lax.axis_index("s") * 2 + lax.axis_index("c")
    base = wid * UPW
    pltpu.sync_copy(eb_hbm.at[pl.ds(base, UPW), :], eb_v)
    pltpu.sync_copy(hist_hbm.at[pl.ds(base, UPW), :], hist_v)

    zf = jnp.zeros((G,), jnp.float32)
    rows = lax.broadcasted_iota(jnp.int32, (G,), 0)

    def zero_row(u, carry):
        def zero_chunk(i, c2):
            obuf[u, pl.ds(jnp.minimum(i * 16, N_ITEMS - 16), 16)] = zf
            return c2
        return lax.fori_loop(0, N_ITEMS // 16 + 1, zero_chunk, carry)

    lax.fori_loop(0, 2 * G, zero_row, 0)

    def hcol(g, j):
        urows = g * G + rows
        return plsc.load_gather(hist_v, [urows, jnp.full((G,), j, jnp.int32)])

    for go in range(NG // 2):
        for b in range(2):
            g = 2 * go + b
            buf = obuf.at[pl.ds(b * G, G), :]
            if go > 0:
                gp = g - 2
                pltpu.make_async_copy(
                    buf, o_hbm.at[pl.ds(base + gp * G, G), :], sems.at[b]
                ).wait()
                for j in range(H):
                    plsc.store_scatter(obuf, [b * G + rows, hcol(gp, j)], zf)
            urows = g * G + rows
            for j in range(H):
                col = hcol(g, j)
                val = plsc.load_gather(
                    eb_v, [urows, jnp.full((G,), j, jnp.int32)]
                )
                plsc.addupdate_scatter(obuf, [b * G + rows, col], val)
            pltpu.async_copy(
                buf, o_hbm.at[pl.ds(base + g * G, G), :], sems.at[b]
            )
    for b in range(2):
        g = NG - 2 + b
        pltpu.make_async_copy(
            obuf.at[pl.ds(b * G, G), :],
            o_hbm.at[pl.ds(base + g * G, G), :],
            sems.at[b],
        ).wait()


def _act_body(o_ref, act_ref, num_ref):
    o = o_ref[...]
    act = o > 0.5
    act_ref[...] = act
    pw = (1 << lax.broadcasted_iota(jnp.int32, (R, H), 1)).astype(jnp.int32)
    num_ref[...] = jnp.sum(act[:, :H].astype(jnp.int32) * pw, axis=1)


def _make_sc_call(interpret=False):
    mesh = plsc.VectorSubcoreMesh(
        core_axis_name="c", subcore_axis_name="s", num_cores=2, num_subcores=16
    )
    return pl.kernel(
        _sc_body,
        out_type=jax.ShapeDtypeStruct((N_USERS, N_ITEMS), jnp.float32),
        mesh=mesh,
        scratch_types=[
            pltpu.VMEM((UPW, 16), jnp.float32),
            pltpu.VMEM((UPW, H), jnp.int32),
            pltpu.VMEM((2 * G, N_ITEMS), jnp.float32),
            pltpu.SemaphoreType.DMA((2,)),
        ],
        compiler_params=pltpu.CompilerParams(needs_layout_passes=False),
        interpret=interpret,
    )


def kernel(distribution, history):
    hist = history.astype(jnp.int32)
    eb = pl.pallas_call(
        _eb_body,
        grid=(N_USERS // R,),
        in_specs=[pl.BlockSpec((R, A), lambda i: (i, 0))],
        out_specs=pl.BlockSpec((R, 16), lambda i: (i, 0)),
        out_shape=jax.ShapeDtypeStruct((N_USERS, 16), jnp.float32),
    )(distribution)
    o = _make_sc_call()(eb, hist)
    act, num = pl.pallas_call(
        _act_body,
        grid=(N_USERS // R,),
        in_specs=[pl.BlockSpec((R, N_ITEMS), lambda i: (i, 0))],
        out_specs=[
            pl.BlockSpec((R, N_ITEMS), lambda i: (i, 0)),
            pl.BlockSpec((R,), lambda i: (i,)),
        ],
        out_shape=[
            jax.ShapeDtypeStruct((N_USERS, N_ITEMS), jnp.bool_),
            jax.ShapeDtypeStruct((N_USERS,), jnp.int32),
        ],
    )(o)
    return (o, act, num)


# speedup vs baseline: 1.2194x; 1.0837x over previous
import jax
import jax.numpy as jnp
from jax import lax
from jax.experimental import pallas as pl
from jax.experimental.pallas import tpu as pltpu
from jax.experimental.pallas import tpu_sc as plsc

N_USERS = 4096
N_ITEMS = 1000
H = 12
A = 1 << H
R = 512
NW = 32
UPW = N_USERS // NW
G = 16
NG = UPW // G


def _eb_body(dist_ref, eb_ref):
    dist = dist_ref[...]
    k_ids = lax.broadcasted_iota(jnp.int32, (A, 128), 0)
    j_ids = jnp.minimum(lax.broadcasted_iota(jnp.int32, (A, 128), 1), 31)
    bitmat = ((k_ids >> j_ids) & 1).astype(jnp.float32)
    eb = jnp.dot(dist, bitmat, preferred_element_type=jnp.float32)
    eb_ref[...] = eb[:, :16]


def _sc_body(eb_hbm, hist_hbm, o_hbm, eb_v, hist_v, obuf):
    wid = lax.axis_index("s") * 2 + lax.axis_index("c")
    base = wid * UPW
    pltpu.sync_copy(eb_hbm.at[pl.ds(base, UPW), :], eb_v)
    pltpu.sync_copy(hist_hbm.at[pl.ds(base, UPW), :], hist_v)

    zf = jnp.zeros((G,), jnp.float32)
    rows = lax.broadcasted_iota(jnp.int32, (G,), 0)

    def zero_row(u, carry):
        def zero_chunk(i, c2):
            obuf[u, pl.ds(jnp.minimum(i * 16, N_ITEMS - 16), 16)] = zf
            return c2
        return lax.fori_loop(0, N_ITEMS // 16 + 1, zero_chunk, carry)

    lax.fori_loop(0, G, zero_row, 0)

    def group(g, carry):
        urows = g * G + rows

        def hcol(j):
            return plsc.load_gather(hist_v, [urows, jnp.full((G,), j, jnp.int32)])

        for j in range(H):
            col = hcol(j)
            val = plsc.load_gather(eb_v, [urows, jnp.full((G,), j, jnp.int32)])
            plsc.addupdate_scatter(obuf, [rows, col], val)
        pltpu.sync_copy(obuf, o_hbm.at[pl.ds(base + g * G, G), :])
        for j in range(H):
            plsc.store_scatter(obuf, [rows, hcol(j)], zf)
        return carry

    lax.fori_loop(0, NG, group, 0)


def _act_body(o_ref, act_ref, num_ref):
    o = o_ref[...]
    act = o > 0.5
    act_ref[...] = act
    pw = (1 << lax.broadcasted_iota(jnp.int32, (R, H), 1)).astype(jnp.int32)
    num_ref[...] = jnp.sum(act[:, :H].astype(jnp.int32) * pw, axis=1)


def _make_sc_call(interpret=False):
    mesh = plsc.VectorSubcoreMesh(
        core_axis_name="c", subcore_axis_name="s", num_cores=2, num_subcores=16
    )
    return pl.kernel(
        _sc_body,
        out_type=jax.ShapeDtypeStruct((N_USERS, N_ITEMS), jnp.float32),
        mesh=mesh,
        scratch_types=[
            pltpu.VMEM((UPW, 16), jnp.float32),
            pltpu.VMEM((UPW, H), jnp.int32),
            pltpu.VMEM((G, N_ITEMS), jnp.float32),
        ],
        compiler_params=pltpu.CompilerParams(needs_layout_passes=False),
        interpret=interpret,
    )


def kernel(distribution, history):
    hist = history.astype(jnp.int32)
    eb = pl.pallas_call(
        _eb_body,
        grid=(N_USERS // R,),
        in_specs=[pl.BlockSpec((R, A), lambda i: (i, 0))],
        out_specs=pl.BlockSpec((R, 16), lambda i: (i, 0)),
        out_shape=jax.ShapeDtypeStruct((N_USERS, 16), jnp.float32),
    )(distribution)
    o = _make_sc_call()(eb, hist)
    act, num = pl.pallas_call(
        _act_body,
        grid=(N_USERS // R,),
        in_specs=[pl.BlockSpec((R, N_ITEMS), lambda i: (i, 0))],
        out_specs=[
            pl.BlockSpec((R, N_ITEMS), lambda i: (i, 0)),
            pl.BlockSpec((R,), lambda i: (i,)),
        ],
        out_shape=[
            jax.ShapeDtypeStruct((N_USERS, N_ITEMS), jnp.bool_),
            jax.ShapeDtypeStruct((N_USERS,), jnp.int32),
        ],
    )(o)
    return (o, act, num)
